# baseline (speedup 1.0000x reference)
import functools

import jax
import jax.numpy as jnp
from jax import lax
from jax.experimental import pallas as pl
from jax.experimental.pallas import tpu as pltpu
from jax.experimental.pallas import tpu_sc as plsc

B, L, D = 64, 512, 768
LANES = 16
NUM_CORES = 2
NUM_SUBCORES = 16
NW = NUM_CORES * NUM_SUBCORES
P = L // NW
COLS = D // LANES

CB = 2
RPC = CB * P
NCHUNK = B // CB
NBUF = 4
AHEAD = 2

ADD_B = 32
NCHUNK_ADD = ADD_B // CB

TB = 4


def _sc_gather_add(tokens_flat, token_table, pos_table):
    mesh = plsc.VectorSubcoreMesh(core_axis_name="c", subcore_axis_name="s")

    scratch = [
        pltpu.VMEM((B * P,), jnp.int32),
        pltpu.VMEM((P, D), jnp.float32),
    ]
    scratch += [pltpu.VMEM((RPC, D), jnp.float32) for _ in range(NBUF)]
    scratch += [pltpu.SemaphoreType.DMA for _ in range(2 * NBUF + 1)]

    @functools.partial(
        pl.kernel,
        out_type=jax.ShapeDtypeStruct((B, L, D), jnp.float32),
        mesh=mesh,
        scratch_types=scratch,
    )
    def k(tokens_hbm, tab_hbm, pos_hbm, out_hbm, idx_v, pos_v, *rest):
        bufs = rest[:NBUF]
        gsem = rest[NBUF:2 * NBUF]
        wsem = rest[2 * NBUF:3 * NBUF]
        ssem = rest[3 * NBUF]

        wid = lax.axis_index("s") * NUM_CORES + lax.axis_index("c")
        p0 = wid * P

        pltpu.sync_copy(pos_hbm.at[pl.ds(p0, P)], pos_v)

        @pl.loop(0, B)
        def stage_idx(b):
            pltpu.async_copy(
                tokens_hbm.at[pl.ds(b * L + p0, P)],
                idx_v.at[pl.ds(b * P, P)], ssem)

        pltpu.make_async_copy(tokens_hbm.at[pl.ds(0, B * P)], idx_v, ssem).wait()

        def fire_gather(t, s):
            pltpu.async_copy(
                tab_hbm.at[idx_v.at[pl.ds(t * RPC, RPC)]], bufs[s], gsem[s])

        for s in range(AHEAD):
            fire_gather(s, s)

        @pl.loop(0, NCHUNK, step=NBUF)
        def outer(t0):
            for s in range(NBUF):
                t = t0 + s
                pltpu.make_async_copy(
                    tab_hbm.at[pl.ds(0, RPC)], bufs[s], gsem[s]).wait()

                @pl.when(t < NCHUNK_ADD)
                def addpos():
                    @pl.loop(0, P)
                    def addrow(r):
                        for c in range(COLS):
                            x = pos_v[r, pl.ds(c * LANES, LANES)]
                            for j in range(CB):
                                sl = (j * P + r, pl.ds(c * LANES, LANES))
                                bufs[s][sl] = bufs[s][sl] + x

                for j in range(CB):
                    pltpu.async_copy(
                        bufs[s].at[pl.ds(j * P, P)],
                        out_hbm.at[t * CB + j, pl.ds(p0, P)], wsem[s])

                tf = t + AHEAD
                sf = (s + AHEAD) % NBUF

                @pl.when(tf < NCHUNK)
                def prefire():
                    @pl.when(tf >= NBUF)
                    def drain_write():
                        pltpu.make_async_copy(
                            tab_hbm.at[pl.ds(0, RPC)], bufs[sf], wsem[sf]
                        ).wait()

                    fire_gather(tf, sf)

        for s in range(NBUF):
            pltpu.make_async_copy(
                tab_hbm.at[pl.ds(0, RPC)], bufs[s], wsem[s]).wait()

    return k(tokens_flat, token_table, pos_table)


def _tc_add_body(g_ref, pos_ref, o_ref):
    o_ref[...] = g_ref[...] + pos_ref[...][None, :, :]


def _tc_fixup(mixed, pos_table):
    base = ADD_B // TB
    return pl.pallas_call(
        _tc_add_body,
        grid=((B - ADD_B) // TB,),
        in_specs=[
            pl.BlockSpec((TB, L, D), lambda i: (i + base, 0, 0)),
            pl.BlockSpec((L, D), lambda i: (0, 0)),
        ],
        out_specs=pl.BlockSpec((TB, L, D), lambda i: (i + base, 0, 0)),
        out_shape=jax.ShapeDtypeStruct((B, L, D), jnp.float32),
        input_output_aliases={0: 0},
    )(mixed, pos_table)


@jax.jit
def _embed(tokens, token_table, pos_table):
    mixed = _sc_gather_add(tokens.reshape(B * L), token_table, pos_table)
    return _tc_fixup(mixed, pos_table)


def kernel(tokens, token_table, pos_table):
    return _embed(tokens, token_table, pos_table)


# speedup vs baseline: 1.0959x; 1.0959x over previous
import functools

import jax
import jax.numpy as jnp
from jax import lax
from jax.experimental import pallas as pl
from jax.experimental.pallas import tpu as pltpu
from jax.experimental.pallas import tpu_sc as plsc

B, L, D = 64, 512, 768
NUM_CORES = 2
NUM_SUBCORES = 16
NW = NUM_CORES * NUM_SUBCORES
P = L // NW

CB = 2
RPC = CB * P
NCHUNK = B // CB
NBUF = 4
AHEAD = 3

TB = 4


def _sc_gather(tokens_flat, token_table):
    mesh = plsc.VectorSubcoreMesh(core_axis_name="c", subcore_axis_name="s")

    scratch = [pltpu.VMEM((B * P,), jnp.int32)]
    scratch += [pltpu.VMEM((RPC, D), jnp.float32) for _ in range(NBUF)]
    scratch += [pltpu.SemaphoreType.DMA for _ in range(2 * NBUF + 1)]

    @functools.partial(
        pl.kernel,
        out_type=jax.ShapeDtypeStruct((B, L, D), jnp.float32),
        mesh=mesh,
        scratch_types=scratch,
    )
    def k(tokens_hbm, tab_hbm, out_hbm, idx_v, *rest):
        bufs = rest[:NBUF]
        gsem = rest[NBUF:2 * NBUF]
        wsem = rest[2 * NBUF:3 * NBUF]
        ssem = rest[3 * NBUF]

        wid = lax.axis_index("s") * NUM_CORES + lax.axis_index("c")
        p0 = wid * P

        @pl.loop(0, B)
        def stage_idx(b):
            pltpu.async_copy(
                tokens_hbm.at[pl.ds(b * L + p0, P)],
                idx_v.at[pl.ds(b * P, P)], ssem)

        pltpu.make_async_copy(tokens_hbm.at[pl.ds(0, B * P)], idx_v, ssem).wait()

        def fire_gather(t, s):
            pltpu.async_copy(
                tab_hbm.at[idx_v.at[pl.ds(t * RPC, RPC)]], bufs[s], gsem[s])

        for s in range(AHEAD):
            fire_gather(s, s)

        @pl.loop(0, NCHUNK, step=NBUF)
        def outer(t0):
            for s in range(NBUF):
                t = t0 + s
                pltpu.make_async_copy(
                    tab_hbm.at[pl.ds(0, RPC)], bufs[s], gsem[s]).wait()
                for j in range(CB):
                    pltpu.async_copy(
                        bufs[s].at[pl.ds(j * P, P)],
                        out_hbm.at[t * CB + j, pl.ds(p0, P)], wsem[s])

                tf = t + AHEAD
                sf = (s + AHEAD) % NBUF

                @pl.when(tf < NCHUNK)
                def prefire():
                    @pl.when(tf >= NBUF)
                    def drain_write():
                        pltpu.make_async_copy(
                            tab_hbm.at[pl.ds(0, RPC)], bufs[sf], wsem[sf]
                        ).wait()

                    fire_gather(tf, sf)

        for s in range(NBUF):
            pltpu.make_async_copy(
                tab_hbm.at[pl.ds(0, RPC)], bufs[s], wsem[s]).wait()

    return k(tokens_flat, token_table)


def _tc_add_body(g_ref, pos_ref, o_ref):
    o_ref[...] = g_ref[...] + pos_ref[...][None, :, :]


def _tc_add(gathered, pos_table):
    return pl.pallas_call(
        _tc_add_body,
        grid=(B // TB,),
        in_specs=[
            pl.BlockSpec((TB, L, D), lambda i: (i, 0, 0)),
            pl.BlockSpec((L, D), lambda i: (0, 0)),
        ],
        out_specs=pl.BlockSpec((TB, L, D), lambda i: (i, 0, 0)),
        out_shape=jax.ShapeDtypeStruct((B, L, D), jnp.float32),
        input_output_aliases={0: 0},
    )(gathered, pos_table)


@jax.jit
def _embed(tokens, token_table, pos_table):
    gathered = _sc_gather(tokens.reshape(B * L), token_table)
    return _tc_add(gathered, pos_table)


def kernel(tokens, token_table, pos_table):
    return _embed(tokens, token_table, pos_table)
